# Initial kernel scaffold; baseline (speedup 1.0000x reference)
import functools

import jax
import jax.numpy as jnp
from jax import lax
from jax.experimental import pallas as pl
from jax.experimental.pallas import tpu as pltpu
from jax.experimental.pallas import tpu_sc as plsc

N = 10000
NP = 10240
E = 320000
D = 128
F = 64
G = 64
GP = 128
NW = 32
EPT = E // NW
CH = 2000
L = 16
TB = 512

_mesh = plsc.VectorSubcoreMesh(core_axis_name="c", subcore_axis_name="s")
_f32 = jnp.float32
_i32 = jnp.int32


def _wid():
    return lax.axis_index("c") * 16 + lax.axis_index("s")


def _deg_body(col_hbm, ew_hbm, parts_hbm, colb, ewb, degp):
    wid = _wid()

    def zero(i, _):
        degp[pl.ds(i * L, L)] = jnp.zeros((L,), _f32)
        return _

    lax.fori_loop(0, NP // L, zero, None)
    base = pl.multiple_of(wid * EPT, 8)

    def chunk(cc, _):
        off = pl.multiple_of(base + cc * CH, 8)
        pltpu.sync_copy(col_hbm.at[pl.ds(off, CH)], colb)
        pltpu.sync_copy(ew_hbm.at[pl.ds(off, CH)], ewb)

        def grp(g, _):
            d = pl.ds(g * L, L)
            plsc.addupdate_scatter(degp, [colb[d]], ewb[d])
            return _

        lax.fori_loop(0, CH // L, grp, None)
        return _

    lax.fori_loop(0, EPT // CH, chunk, None)
    pltpu.sync_copy(degp, parts_hbm.at[wid])


_deg_kernel = pl.kernel(
    _deg_body,
    out_type=jax.ShapeDtypeStruct((NW, NP), _f32),
    mesh=_mesh,
    scratch_types=[
        pltpu.VMEM((CH,), _i32),
        pltpu.VMEM((CH,), _f32),
        pltpu.VMEM((NP,), _f32),
    ],
)


def _norm_body(row_hbm, col_hbm, ew_hbm, dinv_hbm, norm_hbm,
               dinvb, rowb, colb, ewb, normb):
    wid = _wid()
    pltpu.sync_copy(dinv_hbm, dinvb)
    base = pl.multiple_of(wid * EPT, 8)

    def chunk(cc, _):
        off = pl.multiple_of(base + cc * CH, 8)
        pltpu.sync_copy(row_hbm.at[pl.ds(off, CH)], rowb)
        pltpu.sync_copy(col_hbm.at[pl.ds(off, CH)], colb)
        pltpu.sync_copy(ew_hbm.at[pl.ds(off, CH)], ewb)

        def grp(g, _):
            d = pl.ds(g * L, L)
            dr = plsc.load_gather(dinvb, [rowb[d]])
            dc = plsc.load_gather(dinvb, [colb[d]])
            normb[d] = dr * ewb[d] * dc
            return _

        lax.fori_loop(0, CH // L, grp, None)
        pltpu.sync_copy(normb, norm_hbm.at[pl.ds(off, CH)])
        return _

    lax.fori_loop(0, EPT // CH, chunk, None)


_norm_kernel = pl.kernel(
    _norm_body,
    out_type=jax.ShapeDtypeStruct((E,), _f32),
    mesh=_mesh,
    scratch_types=[
        pltpu.VMEM((NP,), _f32),
        pltpu.VMEM((CH,), _i32),
        pltpu.VMEM((CH,), _i32),
        pltpu.VMEM((CH,), _f32),
        pltpu.VMEM((CH,), _f32),
    ],
)


def _agg_body(do_pool, *refs):
    if do_pool:
        (hlin_hbm, hself_hbm, row_hbm, col_hbm, norm_hbm, b_hbm, batch_hbm,
         hout_hbm, pooled_hbm, cnt_hbm,
         hcolA, hcolB, aggA, aggB, rowb, colb, normb, biasb,
         batchb, psA, psB, cntb) = refs
    else:
        (hlin_hbm, hself_hbm, row_hbm, col_hbm, norm_hbm, b_hbm,
         hout_hbm,
         hcolA, hcolB, aggA, aggB, rowb, colb, normb, biasb) = refs

    wid = _wid()
    jA = wid * 2
    jB = jA + 1
    pltpu.sync_copy(hlin_hbm.at[jA], hcolA)
    pltpu.sync_copy(hlin_hbm.at[jB], hcolB)
    pltpu.sync_copy(hself_hbm.at[jA], aggA)
    pltpu.sync_copy(hself_hbm.at[jB], aggB)
    pltpu.sync_copy(b_hbm, biasb)

    def chunk(cc, _):
        off = pl.multiple_of(cc * CH, 8)
        pltpu.sync_copy(row_hbm.at[pl.ds(off, CH)], rowb)
        pltpu.sync_copy(col_hbm.at[pl.ds(off, CH)], colb)
        pltpu.sync_copy(norm_hbm.at[pl.ds(off, CH)], normb)

        def grp(g, _):
            d = pl.ds(g * L, L)
            rv = rowb[d]
            cv = colb[d]
            nv = normb[d]
            plsc.addupdate_scatter(aggA, [cv], plsc.load_gather(hcolA, [rv]) * nv)
            plsc.addupdate_scatter(aggB, [cv], plsc.load_gather(hcolB, [rv]) * nv)
            return _

        lax.fori_loop(0, CH // L, grp, None)
        return _

    lax.fori_loop(0, E // CH, chunk, None)

    bA = plsc.load_gather(biasb, [jnp.full((L,), jA, _i32)])
    bB = plsc.load_gather(biasb, [jnp.full((L,), jB, _i32)])

    def wb(i, _):
        d = pl.ds(i * L, L)
        hcolA[d] = jnp.maximum(aggA[d] + bA, 0.0)
        hcolB[d] = jnp.maximum(aggB[d] + bB, 0.0)
        return _

    lax.fori_loop(0, NP // L, wb, None)
    pltpu.sync_copy(hcolA, hout_hbm.at[jA])
    pltpu.sync_copy(hcolB, hout_hbm.at[jB])

    if do_pool:
        pltpu.sync_copy(batch_hbm, batchb)

        def zero(i, _):
            d = pl.ds(i * L, L)
            psA[d] = jnp.zeros((L,), _f32)
            psB[d] = jnp.zeros((L,), _f32)
            cntb[d] = jnp.zeros((L,), _f32)
            return _

        lax.fori_loop(0, GP // L, zero, None)

        def pool(i, _):
            d = pl.ds(i * L, L)
            bv = batchb[d]
            plsc.addupdate_scatter(psA, [bv], hcolA[d])
            plsc.addupdate_scatter(psB, [bv], hcolB[d])
            return _

        lax.fori_loop(0, NP // L, pool, None)
        pltpu.sync_copy(psA, pooled_hbm.at[jA])
        pltpu.sync_copy(psB, pooled_hbm.at[jB])

        @pl.when(wid == 0)
        def _():
            ones = jnp.ones((L,), _f32)

            def pc(i, _):
                plsc.addupdate_scatter(cntb, [batchb[pl.ds(i * L, L)]], ones)
                return _

            lax.fori_loop(0, NP // L, pc, None)
            pltpu.sync_copy(cntb, cnt_hbm)


_agg_scratch = [
    pltpu.VMEM((NP,), _f32),
    pltpu.VMEM((NP,), _f32),
    pltpu.VMEM((NP,), _f32),
    pltpu.VMEM((NP,), _f32),
    pltpu.VMEM((CH,), _i32),
    pltpu.VMEM((CH,), _i32),
    pltpu.VMEM((CH,), _f32),
    pltpu.VMEM((F,), _f32),
]

_agg_kernel = pl.kernel(
    functools.partial(_agg_body, False),
    out_type=jax.ShapeDtypeStruct((F, NP), _f32),
    mesh=_mesh,
    scratch_types=list(_agg_scratch),
)

_agg_pool_kernel = pl.kernel(
    functools.partial(_agg_body, True),
    out_type=(
        jax.ShapeDtypeStruct((F, NP), _f32),
        jax.ShapeDtypeStruct((F, GP), _f32),
        jax.ShapeDtypeStruct((GP,), _f32),
    ),
    mesh=_mesh,
    scratch_types=list(_agg_scratch) + [
        pltpu.VMEM((NP,), _i32),
        pltpu.VMEM((GP,), _f32),
        pltpu.VMEM((GP,), _f32),
        pltpu.VMEM((GP,), _f32),
    ],
)


def _prep_body(x_ref, w_ref, dp_ref, h1t_ref, hself_ref, dinv_ref, invdeg_ref):
    deg = jnp.sum(dp_ref[...], axis=0, keepdims=True) + 1.0
    dinv = lax.rsqrt(deg)
    invdeg = 1.0 / deg
    ht = lax.dot_general(w_ref[...], x_ref[...], (((0,), (1,)), ((), ())),
                         preferred_element_type=_f32)
    h1t_ref[...] = ht
    hself_ref[...] = ht * invdeg
    dinv_ref[...] = dinv
    invdeg_ref[...] = invdeg


def _mid_body(ht_ref, w_ref, invdeg_ref, hlt_ref, hself_ref):
    hlt = lax.dot_general(w_ref[...], ht_ref[...], (((0,), (0,)), ((), ())),
                          preferred_element_type=_f32)
    hlt_ref[...] = hlt
    hself_ref[...] = hlt * invdeg_ref[...]


def _final_body(pt_ref, cnt_ref, wl_ref, bl_ref, wl2_ref, bl2_ref, out_ref):
    t = lax.dot_general(pt_ref[...], wl_ref[...], (((0,), (0,)), ((), ())),
                        preferred_element_type=_f32)
    t = t / jnp.maximum(cnt_ref[...], 1.0)
    t = jnp.maximum(t + bl_ref[...], 0.0)
    o = lax.dot_general(t, wl2_ref[...], (((1,), (0,)), ((), ())),
                        preferred_element_type=_f32)
    out_ref[...] = o[:G, :] + bl2_ref[...]


def kernel(x, edge_index, edge_weight, batch,
           W1, b1, W2, b2, W3, b3, Wl, bl, Wl2, bl2):
    row = edge_index[0]
    col = edge_index[1]
    xP = jnp.zeros((NP, D), _f32).at[:N].set(x)
    batchP = jnp.concatenate([batch.astype(_i32), jnp.full((NP - N,), G, _i32)])

    deg_parts = _deg_kernel(col, edge_weight)

    nb = NP // TB
    h1t, hself1, dinv2d, invdeg2d = pl.pallas_call(
        _prep_body,
        grid=(nb,),
        in_specs=[
            pl.BlockSpec((TB, D), lambda i: (i, 0)),
            pl.BlockSpec((D, F), lambda i: (0, 0)),
            pl.BlockSpec((NW, TB), lambda i: (0, i)),
        ],
        out_specs=[
            pl.BlockSpec((F, TB), lambda i: (0, i)),
            pl.BlockSpec((F, TB), lambda i: (0, i)),
            pl.BlockSpec((1, TB), lambda i: (0, i)),
            pl.BlockSpec((1, TB), lambda i: (0, i)),
        ],
        out_shape=[
            jax.ShapeDtypeStruct((F, NP), _f32),
            jax.ShapeDtypeStruct((F, NP), _f32),
            jax.ShapeDtypeStruct((1, NP), _f32),
            jax.ShapeDtypeStruct((1, NP), _f32),
        ],
    )(xP, W1, deg_parts)

    norm = _norm_kernel(row, col, edge_weight, jnp.reshape(dinv2d, (NP,)))

    def mid_matmul(ht, W):
        return pl.pallas_call(
            _mid_body,
            grid=(nb,),
            in_specs=[
                pl.BlockSpec((F, TB), lambda i: (0, i)),
                pl.BlockSpec((F, F), lambda i: (0, 0)),
                pl.BlockSpec((1, TB), lambda i: (0, i)),
            ],
            out_specs=[
                pl.BlockSpec((F, TB), lambda i: (0, i)),
                pl.BlockSpec((F, TB), lambda i: (0, i)),
            ],
            out_shape=[
                jax.ShapeDtypeStruct((F, NP), _f32),
                jax.ShapeDtypeStruct((F, NP), _f32),
            ],
        )(ht, W, invdeg2d)

    h2t = _agg_kernel(h1t, hself1, row, col, norm, b1)
    hlt2, hself2 = mid_matmul(h2t, W2)
    h3t = _agg_kernel(hlt2, hself2, row, col, norm, b2)
    hlt3, hself3 = mid_matmul(h3t, W3)
    _, pooled_t, cnt = _agg_pool_kernel(hlt3, hself3, row, col, norm, b3, batchP)

    out = pl.pallas_call(
        _final_body,
        in_specs=[
            pl.BlockSpec((F, GP), lambda: (0, 0)),
            pl.BlockSpec((GP, 1), lambda: (0, 0)),
            pl.BlockSpec((F, 32), lambda: (0, 0)),
            pl.BlockSpec((1, 32), lambda: (0, 0)),
            pl.BlockSpec((32, 1), lambda: (0, 0)),
            pl.BlockSpec((1, 1), lambda: (0, 0)),
        ],
        out_specs=pl.BlockSpec((G, 1), lambda: (0, 0)),
        out_shape=jax.ShapeDtypeStruct((G, 1), _f32),
    )(pooled_t, jnp.reshape(cnt, (GP, 1)), Wl,
      jnp.reshape(bl, (1, 32)), Wl2, jnp.reshape(bl2, (1, 1)))
    return out


# speedup vs baseline: 5.6087x; 5.6087x over previous
import functools

import jax
import jax.numpy as jnp
from jax import lax
from jax.experimental import pallas as pl
from jax.experimental.pallas import tpu as pltpu
from jax.experimental.pallas import tpu_sc as plsc

N = 10000
NP = 10240
E = 320000
D = 128
F = 64
G = 64
GP = 128
NW = 32
EPT = E // NW
CH = 2000
L = 16
TB = 512

_mesh = plsc.VectorSubcoreMesh(core_axis_name="c", subcore_axis_name="s")
_sc_params = pltpu.CompilerParams(needs_layout_passes=False)
_f32 = jnp.float32
_i32 = jnp.int32


def _wid():
    return lax.axis_index("c") * 16 + lax.axis_index("s")


def _deg_body(col_hbm, ew_hbm, parts_hbm, colb, ewb, degp):
    wid = _wid()

    def zero(i, _):
        degp[pl.ds(i * L, L)] = jnp.zeros((L,), _f32)
        return _

    lax.fori_loop(0, NP // L, zero, None)
    base = pl.multiple_of(wid * EPT, 8)

    def chunk(cc, _):
        off = pl.multiple_of(base + cc * CH, 8)
        pltpu.sync_copy(col_hbm.at[pl.ds(off, CH)], colb)
        pltpu.sync_copy(ew_hbm.at[pl.ds(off, CH)], ewb)

        def grp(g, _):
            d = pl.ds(g * L, L)
            plsc.addupdate_scatter(degp, [colb[d]], ewb[d])
            return _

        lax.fori_loop(0, CH // L, grp, None)
        return _

    lax.fori_loop(0, EPT // CH, chunk, None)
    pltpu.sync_copy(degp, parts_hbm.at[wid])


_deg_kernel = pl.kernel(
    _deg_body,
    out_type=jax.ShapeDtypeStruct((NW, NP), _f32),
    mesh=_mesh,
    compiler_params=_sc_params,
    scratch_types=[
        pltpu.VMEM((CH,), _i32),
        pltpu.VMEM((CH,), _f32),
        pltpu.VMEM((NP,), _f32),
    ],
)


def _norm_body(row_hbm, col_hbm, ew_hbm, dinv_hbm, norm_hbm,
               dinvb, rowb, colb, ewb, normb):
    wid = _wid()
    pltpu.sync_copy(dinv_hbm, dinvb)
    base = pl.multiple_of(wid * EPT, 8)

    def chunk(cc, _):
        off = pl.multiple_of(base + cc * CH, 8)
        pltpu.sync_copy(row_hbm.at[pl.ds(off, CH)], rowb)
        pltpu.sync_copy(col_hbm.at[pl.ds(off, CH)], colb)
        pltpu.sync_copy(ew_hbm.at[pl.ds(off, CH)], ewb)

        def grp(g, _):
            d = pl.ds(g * L, L)
            dr = plsc.load_gather(dinvb, [rowb[d]])
            dc = plsc.load_gather(dinvb, [colb[d]])
            normb[d] = dr * ewb[d] * dc
            return _

        lax.fori_loop(0, CH // L, grp, None)
        pltpu.sync_copy(normb, norm_hbm.at[pl.ds(off, CH)])
        return _

    lax.fori_loop(0, EPT // CH, chunk, None)


_norm_kernel = pl.kernel(
    _norm_body,
    out_type=jax.ShapeDtypeStruct((E,), _f32),
    mesh=_mesh,
    compiler_params=_sc_params,
    scratch_types=[
        pltpu.VMEM((NP,), _f32),
        pltpu.VMEM((CH,), _i32),
        pltpu.VMEM((CH,), _i32),
        pltpu.VMEM((CH,), _f32),
        pltpu.VMEM((CH,), _f32),
    ],
)


def _agg_body(do_pool, *refs):
    if do_pool:
        (hlin_hbm, hself_hbm, row_hbm, col_hbm, norm_hbm, b_hbm, batch_hbm,
         hout_hbm, pooled_hbm, cnt_hbm,
         hcolA, hcolB, aggA, aggB, rowb, colb, normb, biasb,
         batchb, psA, psB, cntb) = refs
    else:
        (hlin_hbm, hself_hbm, row_hbm, col_hbm, norm_hbm, b_hbm,
         hout_hbm,
         hcolA, hcolB, aggA, aggB, rowb, colb, normb, biasb) = refs

    wid = _wid()
    jA = wid * 2
    jB = jA + 1
    pltpu.sync_copy(hlin_hbm.at[jA], hcolA)
    pltpu.sync_copy(hlin_hbm.at[jB], hcolB)
    pltpu.sync_copy(hself_hbm.at[jA], aggA)
    pltpu.sync_copy(hself_hbm.at[jB], aggB)
    pltpu.sync_copy(b_hbm, biasb)

    def chunk(cc, _):
        off = pl.multiple_of(cc * CH, 8)
        pltpu.sync_copy(row_hbm.at[pl.ds(off, CH)], rowb)
        pltpu.sync_copy(col_hbm.at[pl.ds(off, CH)], colb)
        pltpu.sync_copy(norm_hbm.at[pl.ds(off, CH)], normb)

        def grp(g, _):
            d = pl.ds(g * L, L)
            rv = rowb[d]
            cv = colb[d]
            nv = normb[d]
            plsc.addupdate_scatter(aggA, [cv], plsc.load_gather(hcolA, [rv]) * nv)
            plsc.addupdate_scatter(aggB, [cv], plsc.load_gather(hcolB, [rv]) * nv)
            return _

        lax.fori_loop(0, CH // L, grp, None)
        return _

    lax.fori_loop(0, E // CH, chunk, None)

    bA = plsc.load_gather(biasb, [jnp.full((L,), jA, _i32)])
    bB = plsc.load_gather(biasb, [jnp.full((L,), jB, _i32)])

    def wb(i, _):
        d = pl.ds(i * L, L)
        hcolA[d] = jnp.maximum(aggA[d] + bA, 0.0)
        hcolB[d] = jnp.maximum(aggB[d] + bB, 0.0)
        return _

    lax.fori_loop(0, NP // L, wb, None)
    pltpu.sync_copy(hcolA, hout_hbm.at[jA])
    pltpu.sync_copy(hcolB, hout_hbm.at[jB])

    if do_pool:
        pltpu.sync_copy(batch_hbm, batchb)

        def zero(i, _):
            d = pl.ds(i * L, L)
            psA[d] = jnp.zeros((L,), _f32)
            psB[d] = jnp.zeros((L,), _f32)
            cntb[d] = jnp.zeros((L,), _f32)
            return _

        lax.fori_loop(0, GP // L, zero, None)

        def pool(i, _):
            d = pl.ds(i * L, L)
            bv = batchb[d]
            plsc.addupdate_scatter(psA, [bv], hcolA[d])
            plsc.addupdate_scatter(psB, [bv], hcolB[d])
            return _

        lax.fori_loop(0, NP // L, pool, None)
        pltpu.sync_copy(psA, pooled_hbm.at[jA])
        pltpu.sync_copy(psB, pooled_hbm.at[jB])

        @pl.when(wid == 0)
        def _():
            ones = jnp.ones((L,), _f32)

            def pc(i, _):
                plsc.addupdate_scatter(cntb, [batchb[pl.ds(i * L, L)]], ones)
                return _

            lax.fori_loop(0, NP // L, pc, None)
            pltpu.sync_copy(cntb, cnt_hbm)


_agg_scratch = [
    pltpu.VMEM((NP,), _f32),
    pltpu.VMEM((NP,), _f32),
    pltpu.VMEM((NP,), _f32),
    pltpu.VMEM((NP,), _f32),
    pltpu.VMEM((CH,), _i32),
    pltpu.VMEM((CH,), _i32),
    pltpu.VMEM((CH,), _f32),
    pltpu.VMEM((F,), _f32),
]

_agg_kernel = pl.kernel(
    functools.partial(_agg_body, False),
    out_type=jax.ShapeDtypeStruct((F, NP), _f32),
    mesh=_mesh,
    compiler_params=_sc_params,
    scratch_types=list(_agg_scratch),
)

_agg_pool_kernel = pl.kernel(
    functools.partial(_agg_body, True),
    out_type=(
        jax.ShapeDtypeStruct((F, NP), _f32),
        jax.ShapeDtypeStruct((F, GP), _f32),
        jax.ShapeDtypeStruct((GP,), _f32),
    ),
    mesh=_mesh,
    compiler_params=_sc_params,
    scratch_types=list(_agg_scratch) + [
        pltpu.VMEM((NP,), _i32),
        pltpu.VMEM((GP,), _f32),
        pltpu.VMEM((GP,), _f32),
        pltpu.VMEM((GP,), _f32),
    ],
)


def _prep_body(x_ref, w_ref, dp_ref, h1t_ref, hself_ref, dinv_ref, invdeg_ref):
    deg = jnp.sum(dp_ref[...], axis=0, keepdims=True) + 1.0
    dinv = lax.rsqrt(deg)
    invdeg = 1.0 / deg
    ht = lax.dot_general(w_ref[...], x_ref[...], (((0,), (1,)), ((), ())),
                         preferred_element_type=_f32)
    h1t_ref[...] = ht
    hself_ref[...] = ht * invdeg
    dinv_ref[...] = dinv
    invdeg_ref[...] = invdeg


def _mid_body(ht_ref, w_ref, invdeg_ref, hlt_ref, hself_ref):
    hlt = lax.dot_general(w_ref[...], ht_ref[...], (((0,), (0,)), ((), ())),
                          preferred_element_type=_f32)
    hlt_ref[...] = hlt
    hself_ref[...] = hlt * invdeg_ref[...]


def _final_body(pt_ref, cnt_ref, wl_ref, bl_ref, wl2_ref, bl2_ref, out_ref):
    t = lax.dot_general(pt_ref[...], wl_ref[...], (((0,), (0,)), ((), ())),
                        preferred_element_type=_f32)
    t = t / jnp.maximum(cnt_ref[...], 1.0)
    t = jnp.maximum(t + bl_ref[...], 0.0)
    o = lax.dot_general(t, wl2_ref[...], (((1,), (0,)), ((), ())),
                        preferred_element_type=_f32)
    out_ref[...] = o[:G, :] + bl2_ref[...]


def kernel(x, edge_index, edge_weight, batch,
           W1, b1, W2, b2, W3, b3, Wl, bl, Wl2, bl2):
    row = edge_index[0]
    col = edge_index[1]
    xP = jnp.zeros((NP, D), _f32).at[:N].set(x)
    batchP = jnp.concatenate([batch.astype(_i32), jnp.full((NP - N,), G, _i32)])

    deg_parts = _deg_kernel(col, edge_weight)

    nb = NP // TB
    h1t, hself1, dinv2d, invdeg2d = pl.pallas_call(
        _prep_body,
        grid=(nb,),
        in_specs=[
            pl.BlockSpec((TB, D), lambda i: (i, 0)),
            pl.BlockSpec((D, F), lambda i: (0, 0)),
            pl.BlockSpec((NW, TB), lambda i: (0, i)),
        ],
        out_specs=[
            pl.BlockSpec((F, TB), lambda i: (0, i)),
            pl.BlockSpec((F, TB), lambda i: (0, i)),
            pl.BlockSpec((1, TB), lambda i: (0, i)),
            pl.BlockSpec((1, TB), lambda i: (0, i)),
        ],
        out_shape=[
            jax.ShapeDtypeStruct((F, NP), _f32),
            jax.ShapeDtypeStruct((F, NP), _f32),
            jax.ShapeDtypeStruct((1, NP), _f32),
            jax.ShapeDtypeStruct((1, NP), _f32),
        ],
    )(xP, W1, deg_parts)

    norm = _norm_kernel(row, col, edge_weight, jnp.reshape(dinv2d, (NP,)))

    def mid_matmul(ht, W):
        return pl.pallas_call(
            _mid_body,
            grid=(nb,),
            in_specs=[
                pl.BlockSpec((F, TB), lambda i: (0, i)),
                pl.BlockSpec((F, F), lambda i: (0, 0)),
                pl.BlockSpec((1, TB), lambda i: (0, i)),
            ],
            out_specs=[
                pl.BlockSpec((F, TB), lambda i: (0, i)),
                pl.BlockSpec((F, TB), lambda i: (0, i)),
            ],
            out_shape=[
                jax.ShapeDtypeStruct((F, NP), _f32),
                jax.ShapeDtypeStruct((F, NP), _f32),
            ],
        )(ht, W, invdeg2d)

    h2t = _agg_kernel(h1t, hself1, row, col, norm, b1)
    hlt2, hself2 = mid_matmul(h2t, W2)
    h3t = _agg_kernel(hlt2, hself2, row, col, norm, b2)
    hlt3, hself3 = mid_matmul(h3t, W3)
    _, pooled_t, cnt = _agg_pool_kernel(hlt3, hself3, row, col, norm, b3, batchP)

    out = pl.pallas_call(
        _final_body,
        in_specs=[
            pl.BlockSpec((F, GP), lambda: (0, 0)),
            pl.BlockSpec((GP, 1), lambda: (0, 0)),
            pl.BlockSpec((F, 32), lambda: (0, 0)),
            pl.BlockSpec((1, 32), lambda: (0, 0)),
            pl.BlockSpec((32, 1), lambda: (0, 0)),
            pl.BlockSpec((1, 1), lambda: (0, 0)),
        ],
        out_specs=pl.BlockSpec((G, 1), lambda: (0, 0)),
        out_shape=jax.ShapeDtypeStruct((G, 1), _f32),
    )(pooled_t, jnp.reshape(cnt, (GP, 1)), Wl,
      jnp.reshape(bl, (1, 32)), Wl2, jnp.reshape(bl2, (1, 1)))
    return out


# speedup vs baseline: 9.8710x; 1.7599x over previous
import functools

import jax
import jax.numpy as jnp
from jax import lax
from jax.experimental import pallas as pl
from jax.experimental.pallas import tpu as pltpu
from jax.experimental.pallas import tpu_sc as plsc

N = 10000
NP = 10240
E = 320000
D = 128
F = 64
G = 64
GP = 128
NW = 32
EPT = E // NW
CH = 2000
ACH = 4000
NCHA = E // ACH
L = 16
TB = 512

_mesh = plsc.VectorSubcoreMesh(core_axis_name="c", subcore_axis_name="s")
_sc_params = pltpu.CompilerParams(needs_layout_passes=False)
_f32 = jnp.float32
_i32 = jnp.int32


def _wid():
    return lax.axis_index("c") * 16 + lax.axis_index("s")


def _deg_body(col_hbm, ew_hbm, parts_hbm, colb, ewb, degp):
    wid = _wid()

    def zero(i, _):
        degp[pl.ds(i * L, L)] = jnp.zeros((L,), _f32)
        return _

    lax.fori_loop(0, NP // L, zero, None)
    base = pl.multiple_of(wid * EPT, 8)

    def chunk(cc, _):
        off = pl.multiple_of(base + cc * CH, 8)
        pltpu.sync_copy(col_hbm.at[pl.ds(off, CH)], colb)
        pltpu.sync_copy(ew_hbm.at[pl.ds(off, CH)], ewb)

        def grp(g, _):
            d = pl.ds(g * L, L)
            plsc.addupdate_scatter(degp, [colb[d]], ewb[d])
            return _

        lax.fori_loop(0, CH // L, grp, None)
        return _

    lax.fori_loop(0, EPT // CH, chunk, None)
    pltpu.sync_copy(degp, parts_hbm.at[wid])


_deg_kernel = pl.kernel(
    _deg_body,
    out_type=jax.ShapeDtypeStruct((NW, NP), _f32),
    mesh=_mesh,
    compiler_params=_sc_params,
    scratch_types=[
        pltpu.VMEM((CH,), _i32),
        pltpu.VMEM((CH,), _f32),
        pltpu.VMEM((NP,), _f32),
    ],
)


def _norm_body(row_hbm, col_hbm, ew_hbm, dinv_hbm, norm_hbm,
               dinvb, rowb, colb, ewb, normb):
    wid = _wid()
    pltpu.sync_copy(dinv_hbm, dinvb)
    base = pl.multiple_of(wid * EPT, 8)

    def chunk(cc, _):
        off = pl.multiple_of(base + cc * CH, 8)
        pltpu.sync_copy(row_hbm.at[pl.ds(off, CH)], rowb)
        pltpu.sync_copy(col_hbm.at[pl.ds(off, CH)], colb)
        pltpu.sync_copy(ew_hbm.at[pl.ds(off, CH)], ewb)

        def grp(g, _):
            d = pl.ds(g * L, L)
            dr = plsc.load_gather(dinvb, [rowb[d]])
            dc = plsc.load_gather(dinvb, [colb[d]])
            normb[d] = dr * ewb[d] * dc
            return _

        lax.fori_loop(0, CH // L, grp, None)
        pltpu.sync_copy(normb, norm_hbm.at[pl.ds(off, CH)])
        return _

    lax.fori_loop(0, EPT // CH, chunk, None)


_norm_kernel = pl.kernel(
    _norm_body,
    out_type=jax.ShapeDtypeStruct((E,), _f32),
    mesh=_mesh,
    compiler_params=_sc_params,
    scratch_types=[
        pltpu.VMEM((NP,), _f32),
        pltpu.VMEM((CH,), _i32),
        pltpu.VMEM((CH,), _i32),
        pltpu.VMEM((CH,), _f32),
        pltpu.VMEM((CH,), _f32),
    ],
)


def _agg_body(do_pool, *refs):
    if do_pool:
        (hlin_hbm, hself_hbm, row_hbm, col_hbm, norm_hbm, b_hbm, batch_hbm,
         hout_hbm, pooled_hbm, cnt_hbm,
         hcolA, hcolB, aggA, aggB,
         rowb0, colb0, normb0, rowb1, colb1, normb1, sem0, sem1, biasb,
         batchb, psA, psB, cntb) = refs
    else:
        (hlin_hbm, hself_hbm, row_hbm, col_hbm, norm_hbm, b_hbm,
         hout_hbm,
         hcolA, hcolB, aggA, aggB,
         rowb0, colb0, normb0, rowb1, colb1, normb1, sem0, sem1, biasb) = refs

    wid = _wid()
    jA = wid * 2
    jB = jA + 1
    pltpu.sync_copy(hlin_hbm.at[jA], hcolA)
    pltpu.sync_copy(hlin_hbm.at[jB], hcolB)
    pltpu.sync_copy(hself_hbm.at[jA], aggA)
    pltpu.sync_copy(hself_hbm.at[jB], aggB)
    pltpu.sync_copy(b_hbm, biasb)

    def issue(c, rowb, colb, normb, sem):
        off = pl.multiple_of(c * ACH, 8)
        pltpu.async_copy(row_hbm.at[pl.ds(off, ACH)], rowb, sem)
        pltpu.async_copy(col_hbm.at[pl.ds(off, ACH)], colb, sem)
        pltpu.async_copy(norm_hbm.at[pl.ds(off, ACH)], normb, sem)

    def wait(rowb, colb, normb, sem):
        pltpu.make_async_copy(row_hbm.at[pl.ds(0, ACH)], rowb, sem).wait()
        pltpu.make_async_copy(col_hbm.at[pl.ds(0, ACH)], colb, sem).wait()
        pltpu.make_async_copy(norm_hbm.at[pl.ds(0, ACH)], normb, sem).wait()

    def compute(rowb, colb, normb):
        def grp(g, _):
            d = pl.ds(g * L, L)
            rv = rowb[d]
            cv = colb[d]
            nv = normb[d]
            plsc.addupdate_scatter(aggA, [cv], plsc.load_gather(hcolA, [rv]) * nv)
            plsc.addupdate_scatter(aggB, [cv], plsc.load_gather(hcolB, [rv]) * nv)
            return _

        lax.fori_loop(0, ACH // L, grp, None, unroll=5)

    issue(0, rowb0, colb0, normb0, sem0)

    def pair(i, _):
        issue(2 * i + 1, rowb1, colb1, normb1, sem1)
        wait(rowb0, colb0, normb0, sem0)
        compute(rowb0, colb0, normb0)

        @pl.when(i < NCHA // 2 - 1)
        def _():
            issue(2 * i + 2, rowb0, colb0, normb0, sem0)

        wait(rowb1, colb1, normb1, sem1)
        compute(rowb1, colb1, normb1)
        return _

    lax.fori_loop(0, NCHA // 2, pair, None)

    bA = plsc.load_gather(biasb, [jnp.full((L,), jA, _i32)])
    bB = plsc.load_gather(biasb, [jnp.full((L,), jB, _i32)])

    def wb(i, _):
        d = pl.ds(i * L, L)
        hcolA[d] = jnp.maximum(aggA[d] + bA, 0.0)
        hcolB[d] = jnp.maximum(aggB[d] + bB, 0.0)
        return _

    lax.fori_loop(0, NP // L, wb, None, unroll=4)
    pltpu.sync_copy(hcolA, hout_hbm.at[jA])
    pltpu.sync_copy(hcolB, hout_hbm.at[jB])

    if do_pool:
        pltpu.sync_copy(batch_hbm, batchb)

        def zero(i, _):
            d = pl.ds(i * L, L)
            psA[d] = jnp.zeros((L,), _f32)
            psB[d] = jnp.zeros((L,), _f32)
            cntb[d] = jnp.zeros((L,), _f32)
            return _

        lax.fori_loop(0, GP // L, zero, None)

        def pool(i, _):
            d = pl.ds(i * L, L)
            bv = batchb[d]
            plsc.addupdate_scatter(psA, [bv], hcolA[d])
            plsc.addupdate_scatter(psB, [bv], hcolB[d])
            return _

        lax.fori_loop(0, NP // L, pool, None)
        pltpu.sync_copy(psA, pooled_hbm.at[jA])
        pltpu.sync_copy(psB, pooled_hbm.at[jB])

        @pl.when(wid == 0)
        def _():
            ones = jnp.ones((L,), _f32)

            def pc(i, _):
                plsc.addupdate_scatter(cntb, [batchb[pl.ds(i * L, L)]], ones)
                return _

            lax.fori_loop(0, NP // L, pc, None)
            pltpu.sync_copy(cntb, cnt_hbm)


_agg_scratch = [
    pltpu.VMEM((NP,), _f32),
    pltpu.VMEM((NP,), _f32),
    pltpu.VMEM((NP,), _f32),
    pltpu.VMEM((NP,), _f32),
    pltpu.VMEM((ACH,), _i32),
    pltpu.VMEM((ACH,), _i32),
    pltpu.VMEM((ACH,), _f32),
    pltpu.VMEM((ACH,), _i32),
    pltpu.VMEM((ACH,), _i32),
    pltpu.VMEM((ACH,), _f32),
    pltpu.SemaphoreType.DMA,
    pltpu.SemaphoreType.DMA,
    pltpu.VMEM((F,), _f32),
]

_agg_kernel = pl.kernel(
    functools.partial(_agg_body, False),
    out_type=jax.ShapeDtypeStruct((F, NP), _f32),
    mesh=_mesh,
    compiler_params=_sc_params,
    scratch_types=list(_agg_scratch),
)

_agg_pool_kernel = pl.kernel(
    functools.partial(_agg_body, True),
    out_type=(
        jax.ShapeDtypeStruct((F, NP), _f32),
        jax.ShapeDtypeStruct((F, GP), _f32),
        jax.ShapeDtypeStruct((GP,), _f32),
    ),
    mesh=_mesh,
    compiler_params=_sc_params,
    scratch_types=list(_agg_scratch) + [
        pltpu.VMEM((NP,), _i32),
        pltpu.VMEM((GP,), _f32),
        pltpu.VMEM((GP,), _f32),
        pltpu.VMEM((GP,), _f32),
    ],
)


def _prep_body(x_ref, w_ref, dp_ref, h1t_ref, hself_ref, dinv_ref, invdeg_ref):
    deg = jnp.sum(dp_ref[...], axis=0, keepdims=True) + 1.0
    dinv = lax.rsqrt(deg)
    invdeg = 1.0 / deg
    ht = lax.dot_general(w_ref[...], x_ref[...], (((0,), (1,)), ((), ())),
                         preferred_element_type=_f32)
    h1t_ref[...] = ht
    hself_ref[...] = ht * invdeg
    dinv_ref[...] = dinv
    invdeg_ref[...] = invdeg


def _mid_body(ht_ref, w_ref, invdeg_ref, hlt_ref, hself_ref):
    hlt = lax.dot_general(w_ref[...], ht_ref[...], (((0,), (0,)), ((), ())),
                          preferred_element_type=_f32)
    hlt_ref[...] = hlt
    hself_ref[...] = hlt * invdeg_ref[...]


def _final_body(pt_ref, cnt_ref, wl_ref, bl_ref, wl2_ref, bl2_ref, out_ref):
    t = lax.dot_general(pt_ref[...], wl_ref[...], (((0,), (0,)), ((), ())),
                        preferred_element_type=_f32)
    t = t / jnp.maximum(cnt_ref[...], 1.0)
    t = jnp.maximum(t + bl_ref[...], 0.0)
    o = lax.dot_general(t, wl2_ref[...], (((1,), (0,)), ((), ())),
                        preferred_element_type=_f32)
    out_ref[...] = o[:G, :] + bl2_ref[...]


def kernel(x, edge_index, edge_weight, batch,
           W1, b1, W2, b2, W3, b3, Wl, bl, Wl2, bl2):
    row = edge_index[0]
    col = edge_index[1]
    xP = jnp.zeros((NP, D), _f32).at[:N].set(x)
    batchP = jnp.concatenate([batch.astype(_i32), jnp.full((NP - N,), G, _i32)])

    deg_parts = _deg_kernel(col, edge_weight)

    nb = NP // TB
    h1t, hself1, dinv2d, invdeg2d = pl.pallas_call(
        _prep_body,
        grid=(nb,),
        in_specs=[
            pl.BlockSpec((TB, D), lambda i: (i, 0)),
            pl.BlockSpec((D, F), lambda i: (0, 0)),
            pl.BlockSpec((NW, TB), lambda i: (0, i)),
        ],
        out_specs=[
            pl.BlockSpec((F, TB), lambda i: (0, i)),
            pl.BlockSpec((F, TB), lambda i: (0, i)),
            pl.BlockSpec((1, TB), lambda i: (0, i)),
            pl.BlockSpec((1, TB), lambda i: (0, i)),
        ],
        out_shape=[
            jax.ShapeDtypeStruct((F, NP), _f32),
            jax.ShapeDtypeStruct((F, NP), _f32),
            jax.ShapeDtypeStruct((1, NP), _f32),
            jax.ShapeDtypeStruct((1, NP), _f32),
        ],
    )(xP, W1, deg_parts)

    norm = _norm_kernel(row, col, edge_weight, jnp.reshape(dinv2d, (NP,)))

    def mid_matmul(ht, W):
        return pl.pallas_call(
            _mid_body,
            grid=(nb,),
            in_specs=[
                pl.BlockSpec((F, TB), lambda i: (0, i)),
                pl.BlockSpec((F, F), lambda i: (0, 0)),
                pl.BlockSpec((1, TB), lambda i: (0, i)),
            ],
            out_specs=[
                pl.BlockSpec((F, TB), lambda i: (0, i)),
                pl.BlockSpec((F, TB), lambda i: (0, i)),
            ],
            out_shape=[
                jax.ShapeDtypeStruct((F, NP), _f32),
                jax.ShapeDtypeStruct((F, NP), _f32),
            ],
        )(ht, W, invdeg2d)

    h2t = _agg_kernel(h1t, hself1, row, col, norm, b1)
    hlt2, hself2 = mid_matmul(h2t, W2)
    h3t = _agg_kernel(hlt2, hself2, row, col, norm, b2)
    hlt3, hself3 = mid_matmul(h3t, W3)
    _, pooled_t, cnt = _agg_pool_kernel(hlt3, hself3, row, col, norm, b3, batchP)

    out = pl.pallas_call(
        _final_body,
        in_specs=[
            pl.BlockSpec((F, GP), lambda: (0, 0)),
            pl.BlockSpec((GP, 1), lambda: (0, 0)),
            pl.BlockSpec((F, 32), lambda: (0, 0)),
            pl.BlockSpec((1, 32), lambda: (0, 0)),
            pl.BlockSpec((32, 1), lambda: (0, 0)),
            pl.BlockSpec((1, 1), lambda: (0, 0)),
        ],
        out_specs=pl.BlockSpec((G, 1), lambda: (0, 0)),
        out_shape=jax.ShapeDtypeStruct((G, 1), _f32),
    )(pooled_t, jnp.reshape(cnt, (GP, 1)), Wl,
      jnp.reshape(bl, (1, 32)), Wl2, jnp.reshape(bl2, (1, 1)))
    return out


# speedup vs baseline: 20.6835x; 2.0954x over previous
import functools

import jax
import jax.numpy as jnp
from jax import lax
from jax.experimental import pallas as pl
from jax.experimental.pallas import tpu as pltpu
from jax.experimental.pallas import tpu_sc as plsc

N = 10000
NP = 10240
E = 320000
D = 128
F = 64
G = 64
GP = 128
NW = 32
EPT = E // NW
CH = 2000
ACH = 4000
NCHA = E // ACH
L = 16
TB = 512

_mesh = plsc.VectorSubcoreMesh(core_axis_name="c", subcore_axis_name="s")
_sc_params = pltpu.CompilerParams(needs_layout_passes=False)
_f32 = jnp.float32
_i32 = jnp.int32


def _wid():
    return lax.axis_index("c") * 16 + lax.axis_index("s")


def _deg_body(col_hbm, ew_hbm, parts_hbm, colb, ewb, degp):
    wid = _wid()

    def zero(i, _):
        degp[pl.ds(i * L, L)] = jnp.zeros((L,), _f32)
        return _

    lax.fori_loop(0, NP // L, zero, None)
    base = pl.multiple_of(wid * EPT, 8)

    def chunk(cc, _):
        off = pl.multiple_of(base + cc * CH, 8)
        pltpu.sync_copy(col_hbm.at[pl.ds(off, CH)], colb)
        pltpu.sync_copy(ew_hbm.at[pl.ds(off, CH)], ewb)

        def grp(g, _):
            d = pl.ds(g * L, L)
            plsc.addupdate_scatter(degp, [colb[d]], ewb[d])
            return _

        lax.fori_loop(0, CH // L, grp, None)
        return _

    lax.fori_loop(0, EPT // CH, chunk, None)
    pltpu.sync_copy(degp, parts_hbm.at[wid])


_deg_kernel = pl.kernel(
    _deg_body,
    out_type=jax.ShapeDtypeStruct((NW, NP), _f32),
    mesh=_mesh,
    compiler_params=_sc_params,
    scratch_types=[
        pltpu.VMEM((CH,), _i32),
        pltpu.VMEM((CH,), _f32),
        pltpu.VMEM((NP,), _f32),
    ],
)


def _norm_body(row_hbm, col_hbm, ew_hbm, dinv_hbm, norm_hbm,
               dinvb, rowb, colb, ewb, normb):
    wid = _wid()
    pltpu.sync_copy(dinv_hbm, dinvb)
    base = pl.multiple_of(wid * EPT, 8)

    def chunk(cc, _):
        off = pl.multiple_of(base + cc * CH, 8)
        pltpu.sync_copy(row_hbm.at[pl.ds(off, CH)], rowb)
        pltpu.sync_copy(col_hbm.at[pl.ds(off, CH)], colb)
        pltpu.sync_copy(ew_hbm.at[pl.ds(off, CH)], ewb)

        def grp(g, _):
            d = pl.ds(g * L, L)
            dr = plsc.load_gather(dinvb, [rowb[d]])
            dc = plsc.load_gather(dinvb, [colb[d]])
            normb[d] = dr * ewb[d] * dc
            return _

        lax.fori_loop(0, CH // L, grp, None)
        pltpu.sync_copy(normb, norm_hbm.at[pl.ds(off, CH)])
        return _

    lax.fori_loop(0, EPT // CH, chunk, None)


_norm_kernel = pl.kernel(
    _norm_body,
    out_type=jax.ShapeDtypeStruct((E,), _f32),
    mesh=_mesh,
    compiler_params=_sc_params,
    scratch_types=[
        pltpu.VMEM((NP,), _f32),
        pltpu.VMEM((CH,), _i32),
        pltpu.VMEM((CH,), _i32),
        pltpu.VMEM((CH,), _f32),
        pltpu.VMEM((CH,), _f32),
    ],
)


def _agg_body(do_pool, *refs):
    if do_pool:
        (hlin_hbm, hself_hbm, row_hbm, col_hbm, norm_hbm, b_hbm, batch_hbm,
         hout_hbm, pooled_hbm, cnt_hbm,
         hcolA, hcolB, aggA, aggB,
         rowb0, colb0, normb0, rowb1, colb1, normb1, sem0, sem1, biasb,
         batchb, psA, psB, cntb) = refs
    else:
        (hlin_hbm, hself_hbm, row_hbm, col_hbm, norm_hbm, b_hbm,
         hout_hbm,
         hcolA, hcolB, aggA, aggB,
         rowb0, colb0, normb0, rowb1, colb1, normb1, sem0, sem1, biasb) = refs

    wid = _wid()
    jA = wid * 2
    jB = jA + 1
    pltpu.sync_copy(hlin_hbm.at[jA], hcolA)
    pltpu.sync_copy(hlin_hbm.at[jB], hcolB)
    pltpu.sync_copy(hself_hbm.at[jA], aggA)
    pltpu.sync_copy(hself_hbm.at[jB], aggB)
    pltpu.sync_copy(b_hbm, biasb)

    def issue(c, rowb, colb, normb, sem):
        off = pl.multiple_of(c * ACH, 8)
        pltpu.async_copy(row_hbm.at[pl.ds(off, ACH)], rowb, sem)
        pltpu.async_copy(col_hbm.at[pl.ds(off, ACH)], colb, sem)
        pltpu.async_copy(norm_hbm.at[pl.ds(off, ACH)], normb, sem)

    def wait(rowb, colb, normb, sem):
        pltpu.make_async_copy(row_hbm.at[pl.ds(0, ACH)], rowb, sem).wait()
        pltpu.make_async_copy(col_hbm.at[pl.ds(0, ACH)], colb, sem).wait()
        pltpu.make_async_copy(norm_hbm.at[pl.ds(0, ACH)], normb, sem).wait()

    def compute(rowb, colb, normb):
        @plsc.parallel_loop(0, ACH // L, 1, unroll=5)
        def _(g):
            d = pl.ds(g * L, L)
            rv = rowb[d]
            cv = colb[d]
            nv = normb[d]
            plsc.addupdate_scatter(aggA, [cv], plsc.load_gather(hcolA, [rv]) * nv)
            plsc.addupdate_scatter(aggB, [cv], plsc.load_gather(hcolB, [rv]) * nv)

    issue(0, rowb0, colb0, normb0, sem0)

    def pair(i, _):
        issue(2 * i + 1, rowb1, colb1, normb1, sem1)
        wait(rowb0, colb0, normb0, sem0)
        compute(rowb0, colb0, normb0)

        @pl.when(i < NCHA // 2 - 1)
        def _():
            issue(2 * i + 2, rowb0, colb0, normb0, sem0)

        wait(rowb1, colb1, normb1, sem1)
        compute(rowb1, colb1, normb1)
        return _

    lax.fori_loop(0, NCHA // 2, pair, None)

    bA = plsc.load_gather(biasb, [jnp.full((L,), jA, _i32)])
    bB = plsc.load_gather(biasb, [jnp.full((L,), jB, _i32)])

    def wb(i, _):
        d = pl.ds(i * L, L)
        hcolA[d] = jnp.maximum(aggA[d] + bA, 0.0)
        hcolB[d] = jnp.maximum(aggB[d] + bB, 0.0)
        return _

    lax.fori_loop(0, NP // L, wb, None, unroll=4)
    pltpu.sync_copy(hcolA, hout_hbm.at[jA])
    pltpu.sync_copy(hcolB, hout_hbm.at[jB])

    if do_pool:
        pltpu.sync_copy(batch_hbm, batchb)

        def zero(i, _):
            d = pl.ds(i * L, L)
            psA[d] = jnp.zeros((L,), _f32)
            psB[d] = jnp.zeros((L,), _f32)
            cntb[d] = jnp.zeros((L,), _f32)
            return _

        lax.fori_loop(0, GP // L, zero, None)

        def pool(i, _):
            d = pl.ds(i * L, L)
            bv = batchb[d]
            plsc.addupdate_scatter(psA, [bv], hcolA[d])
            plsc.addupdate_scatter(psB, [bv], hcolB[d])
            return _

        lax.fori_loop(0, NP // L, pool, None)
        pltpu.sync_copy(psA, pooled_hbm.at[jA])
        pltpu.sync_copy(psB, pooled_hbm.at[jB])

        @pl.when(wid == 0)
        def _():
            ones = jnp.ones((L,), _f32)

            def pc(i, _):
                plsc.addupdate_scatter(cntb, [batchb[pl.ds(i * L, L)]], ones)
                return _

            lax.fori_loop(0, NP // L, pc, None)
            pltpu.sync_copy(cntb, cnt_hbm)


_agg_scratch = [
    pltpu.VMEM((NP,), _f32),
    pltpu.VMEM((NP,), _f32),
    pltpu.VMEM((NP,), _f32),
    pltpu.VMEM((NP,), _f32),
    pltpu.VMEM((ACH,), _i32),
    pltpu.VMEM((ACH,), _i32),
    pltpu.VMEM((ACH,), _f32),
    pltpu.VMEM((ACH,), _i32),
    pltpu.VMEM((ACH,), _i32),
    pltpu.VMEM((ACH,), _f32),
    pltpu.SemaphoreType.DMA,
    pltpu.SemaphoreType.DMA,
    pltpu.VMEM((F,), _f32),
]

_agg_kernel = pl.kernel(
    functools.partial(_agg_body, False),
    out_type=jax.ShapeDtypeStruct((F, NP), _f32),
    mesh=_mesh,
    compiler_params=_sc_params,
    scratch_types=list(_agg_scratch),
)

_agg_pool_kernel = pl.kernel(
    functools.partial(_agg_body, True),
    out_type=(
        jax.ShapeDtypeStruct((F, NP), _f32),
        jax.ShapeDtypeStruct((F, GP), _f32),
        jax.ShapeDtypeStruct((GP,), _f32),
    ),
    mesh=_mesh,
    compiler_params=_sc_params,
    scratch_types=list(_agg_scratch) + [
        pltpu.VMEM((NP,), _i32),
        pltpu.VMEM((GP,), _f32),
        pltpu.VMEM((GP,), _f32),
        pltpu.VMEM((GP,), _f32),
    ],
)


def _prep_body(x_ref, w_ref, dp_ref, h1t_ref, hself_ref, dinv_ref, invdeg_ref):
    deg = jnp.sum(dp_ref[...], axis=0, keepdims=True) + 1.0
    dinv = lax.rsqrt(deg)
    invdeg = 1.0 / deg
    ht = lax.dot_general(w_ref[...], x_ref[...], (((0,), (1,)), ((), ())),
                         preferred_element_type=_f32)
    h1t_ref[...] = ht
    hself_ref[...] = ht * invdeg
    dinv_ref[...] = dinv
    invdeg_ref[...] = invdeg


def _mid_body(ht_ref, w_ref, invdeg_ref, hlt_ref, hself_ref):
    hlt = lax.dot_general(w_ref[...], ht_ref[...], (((0,), (0,)), ((), ())),
                          preferred_element_type=_f32)
    hlt_ref[...] = hlt
    hself_ref[...] = hlt * invdeg_ref[...]


def _final_body(pt_ref, cnt_ref, wl_ref, bl_ref, wl2_ref, bl2_ref, out_ref):
    t = lax.dot_general(pt_ref[...], wl_ref[...], (((0,), (0,)), ((), ())),
                        preferred_element_type=_f32)
    t = t / jnp.maximum(cnt_ref[...], 1.0)
    t = jnp.maximum(t + bl_ref[...], 0.0)
    o = lax.dot_general(t, wl2_ref[...], (((1,), (0,)), ((), ())),
                        preferred_element_type=_f32)
    out_ref[...] = o[:G, :] + bl2_ref[...]


def kernel(x, edge_index, edge_weight, batch,
           W1, b1, W2, b2, W3, b3, Wl, bl, Wl2, bl2):
    row = edge_index[0]
    col = edge_index[1]
    xP = jnp.zeros((NP, D), _f32).at[:N].set(x)
    batchP = jnp.concatenate([batch.astype(_i32), jnp.full((NP - N,), G, _i32)])

    deg_parts = _deg_kernel(col, edge_weight)

    nb = NP // TB
    h1t, hself1, dinv2d, invdeg2d = pl.pallas_call(
        _prep_body,
        grid=(nb,),
        in_specs=[
            pl.BlockSpec((TB, D), lambda i: (i, 0)),
            pl.BlockSpec((D, F), lambda i: (0, 0)),
            pl.BlockSpec((NW, TB), lambda i: (0, i)),
        ],
        out_specs=[
            pl.BlockSpec((F, TB), lambda i: (0, i)),
            pl.BlockSpec((F, TB), lambda i: (0, i)),
            pl.BlockSpec((1, TB), lambda i: (0, i)),
            pl.BlockSpec((1, TB), lambda i: (0, i)),
        ],
        out_shape=[
            jax.ShapeDtypeStruct((F, NP), _f32),
            jax.ShapeDtypeStruct((F, NP), _f32),
            jax.ShapeDtypeStruct((1, NP), _f32),
            jax.ShapeDtypeStruct((1, NP), _f32),
        ],
    )(xP, W1, deg_parts)

    norm = _norm_kernel(row, col, edge_weight, jnp.reshape(dinv2d, (NP,)))

    def mid_matmul(ht, W):
        return pl.pallas_call(
            _mid_body,
            grid=(nb,),
            in_specs=[
                pl.BlockSpec((F, TB), lambda i: (0, i)),
                pl.BlockSpec((F, F), lambda i: (0, 0)),
                pl.BlockSpec((1, TB), lambda i: (0, i)),
            ],
            out_specs=[
                pl.BlockSpec((F, TB), lambda i: (0, i)),
                pl.BlockSpec((F, TB), lambda i: (0, i)),
            ],
            out_shape=[
                jax.ShapeDtypeStruct((F, NP), _f32),
                jax.ShapeDtypeStruct((F, NP), _f32),
            ],
        )(ht, W, invdeg2d)

    h2t = _agg_kernel(h1t, hself1, row, col, norm, b1)
    hlt2, hself2 = mid_matmul(h2t, W2)
    h3t = _agg_kernel(hlt2, hself2, row, col, norm, b2)
    hlt3, hself3 = mid_matmul(h3t, W3)
    _, pooled_t, cnt = _agg_pool_kernel(hlt3, hself3, row, col, norm, b3, batchP)

    out = pl.pallas_call(
        _final_body,
        in_specs=[
            pl.BlockSpec((F, GP), lambda: (0, 0)),
            pl.BlockSpec((GP, 1), lambda: (0, 0)),
            pl.BlockSpec((F, 32), lambda: (0, 0)),
            pl.BlockSpec((1, 32), lambda: (0, 0)),
            pl.BlockSpec((32, 1), lambda: (0, 0)),
            pl.BlockSpec((1, 1), lambda: (0, 0)),
        ],
        out_specs=pl.BlockSpec((G, 1), lambda: (0, 0)),
        out_shape=jax.ShapeDtypeStruct((G, 1), _f32),
    )(pooled_t, jnp.reshape(cnt, (GP, 1)), Wl,
      jnp.reshape(bl, (1, 32)), Wl2, jnp.reshape(bl2, (1, 1)))
    return out


# speedup vs baseline: 24.3445x; 1.1770x over previous
import functools

import jax
import jax.numpy as jnp
from jax import lax
from jax.experimental import pallas as pl
from jax.experimental.pallas import tpu as pltpu
from jax.experimental.pallas import tpu_sc as plsc

N = 10000
NP = 10240
E = 320000
D = 128
F = 64
G = 64
GP = 128
NW = 32
EPT = E // NW
CH = 2000
ACH = 4000
NCHA = E // ACH
L = 16
TB = 512

_mesh = plsc.VectorSubcoreMesh(core_axis_name="c", subcore_axis_name="s")
_sc_params = pltpu.CompilerParams(needs_layout_passes=False)
_f32 = jnp.float32
_i32 = jnp.int32


def _wid():
    return lax.axis_index("c") * 16 + lax.axis_index("s")


def _deg_body(col_hbm, ew_hbm, parts_hbm, colb, ewb, degp):
    wid = _wid()

    def zero(i, _):
        degp[pl.ds(i * L, L)] = jnp.zeros((L,), _f32)
        return _

    lax.fori_loop(0, NP // L, zero, None)
    base = pl.multiple_of(wid * EPT, 8)

    def chunk(cc, _):
        off = pl.multiple_of(base + cc * CH, 8)
        pltpu.sync_copy(col_hbm.at[pl.ds(off, CH)], colb)
        pltpu.sync_copy(ew_hbm.at[pl.ds(off, CH)], ewb)

        @plsc.parallel_loop(0, CH // L, 1, unroll=5)
        def _(g):
            d = pl.ds(g * L, L)
            plsc.addupdate_scatter(degp, [colb[d]], ewb[d])

        return _

    lax.fori_loop(0, EPT // CH, chunk, None)
    pltpu.sync_copy(degp, parts_hbm.at[wid])


_deg_kernel = pl.kernel(
    _deg_body,
    out_type=jax.ShapeDtypeStruct((NW, NP), _f32),
    mesh=_mesh,
    compiler_params=_sc_params,
    scratch_types=[
        pltpu.VMEM((CH,), _i32),
        pltpu.VMEM((CH,), _f32),
        pltpu.VMEM((NP,), _f32),
    ],
)


def _norm_body(row_hbm, col_hbm, ew_hbm, dinv_hbm, norm_hbm, pk_hbm,
               dinvb, rowb, colb, ewb, normb, packb):
    wid = _wid()
    pltpu.sync_copy(dinv_hbm, dinvb)
    base = pl.multiple_of(wid * EPT, 8)

    def chunk(cc, _):
        off = pl.multiple_of(base + cc * CH, 8)
        pltpu.sync_copy(row_hbm.at[pl.ds(off, CH)], rowb)
        pltpu.sync_copy(col_hbm.at[pl.ds(off, CH)], colb)
        pltpu.sync_copy(ew_hbm.at[pl.ds(off, CH)], ewb)

        @plsc.parallel_loop(0, CH // L, 1, unroll=5)
        def _(g):
            d = pl.ds(g * L, L)
            rv = rowb[d]
            cv = colb[d]
            dr = plsc.load_gather(dinvb, [rv])
            dc = plsc.load_gather(dinvb, [cv])
            normb[d] = dr * ewb[d] * dc
            packb[d] = jnp.bitwise_or(lax.shift_left(rv, 16), cv)

        pltpu.sync_copy(normb, norm_hbm.at[pl.ds(off, CH)])
        pltpu.sync_copy(packb, pk_hbm.at[pl.ds(off, CH)])
        return _

    lax.fori_loop(0, EPT // CH, chunk, None)


_norm_kernel = pl.kernel(
    _norm_body,
    out_type=(
        jax.ShapeDtypeStruct((E,), _f32),
        jax.ShapeDtypeStruct((E,), _i32),
    ),
    mesh=_mesh,
    compiler_params=_sc_params,
    scratch_types=[
        pltpu.VMEM((NP,), _f32),
        pltpu.VMEM((CH,), _i32),
        pltpu.VMEM((CH,), _i32),
        pltpu.VMEM((CH,), _f32),
        pltpu.VMEM((CH,), _f32),
        pltpu.VMEM((CH,), _i32),
    ],
)


def _agg_body(hlin_hbm, hself_hbm, pk_hbm, norm_hbm, b_hbm,
              hout_hbm,
              hcolA, hcolB, aggA, aggB,
              pkb0, normb0, pkb1, normb1, sem0, sem1, biasb):
    wid = _wid()
    jA = wid * 2
    jB = jA + 1
    pltpu.sync_copy(hlin_hbm.at[jA], hcolA)
    pltpu.sync_copy(hlin_hbm.at[jB], hcolB)
    pltpu.sync_copy(hself_hbm.at[jA], aggA)
    pltpu.sync_copy(hself_hbm.at[jB], aggB)
    pltpu.sync_copy(b_hbm, biasb)

    def issue(c, pkb, normb, sem):
        off = pl.multiple_of(c * ACH, 8)
        pltpu.async_copy(pk_hbm.at[pl.ds(off, ACH)], pkb, sem)
        pltpu.async_copy(norm_hbm.at[pl.ds(off, ACH)], normb, sem)

    def wait(pkb, normb, sem):
        pltpu.make_async_copy(pk_hbm.at[pl.ds(0, ACH)], pkb, sem).wait()
        pltpu.make_async_copy(norm_hbm.at[pl.ds(0, ACH)], normb, sem).wait()

    def compute(pkb, normb):
        @plsc.parallel_loop(0, ACH // L, 1, unroll=5)
        def _(g):
            d = pl.ds(g * L, L)
            pk = pkb[d]
            rv = lax.shift_right_logical(pk, 16)
            cv = jnp.bitwise_and(pk, 0xFFFF)
            nv = normb[d]
            plsc.addupdate_scatter(aggA, [cv], plsc.load_gather(hcolA, [rv]) * nv)
            plsc.addupdate_scatter(aggB, [cv], plsc.load_gather(hcolB, [rv]) * nv)

    issue(0, pkb0, normb0, sem0)

    def pair(i, _):
        issue(2 * i + 1, pkb1, normb1, sem1)
        wait(pkb0, normb0, sem0)
        compute(pkb0, normb0)

        @pl.when(i < NCHA // 2 - 1)
        def _():
            issue(2 * i + 2, pkb0, normb0, sem0)

        wait(pkb1, normb1, sem1)
        compute(pkb1, normb1)
        return _

    lax.fori_loop(0, NCHA // 2, pair, None)

    bA = plsc.load_gather(biasb, [jnp.full((L,), jA, _i32)])
    bB = plsc.load_gather(biasb, [jnp.full((L,), jB, _i32)])

    def wb(i, _):
        d = pl.ds(i * L, L)
        hcolA[d] = jnp.maximum(aggA[d] + bA, 0.0)
        hcolB[d] = jnp.maximum(aggB[d] + bB, 0.0)
        return _

    lax.fori_loop(0, NP // L, wb, None, unroll=4)
    pltpu.sync_copy(hcolA, hout_hbm.at[jA])
    pltpu.sync_copy(hcolB, hout_hbm.at[jB])


_agg_kernel = pl.kernel(
    _agg_body,
    out_type=jax.ShapeDtypeStruct((F, NP), _f32),
    mesh=_mesh,
    compiler_params=_sc_params,
    scratch_types=[
        pltpu.VMEM((NP,), _f32),
        pltpu.VMEM((NP,), _f32),
        pltpu.VMEM((NP,), _f32),
        pltpu.VMEM((NP,), _f32),
        pltpu.VMEM((ACH,), _i32),
        pltpu.VMEM((ACH,), _f32),
        pltpu.VMEM((ACH,), _i32),
        pltpu.VMEM((ACH,), _f32),
        pltpu.SemaphoreType.DMA,
        pltpu.SemaphoreType.DMA,
        pltpu.VMEM((F,), _f32),
    ],
)


def _prep_body(x_ref, w_ref, dp_ref, h1t_ref, hself_ref, dinv_ref, invdeg_ref):
    deg = jnp.sum(dp_ref[...], axis=0, keepdims=True) + 1.0
    dinv = 1.0 / jnp.sqrt(deg)
    invdeg = dinv * dinv
    ht = lax.dot_general(w_ref[...].astype(jnp.bfloat16),
                         x_ref[...].astype(jnp.bfloat16),
                         (((0,), (1,)), ((), ())),
                         preferred_element_type=_f32)
    h1t_ref[...] = ht
    hself_ref[...] = ht * invdeg
    dinv_ref[...] = dinv
    invdeg_ref[...] = invdeg


def _mid_body(ht_ref, w_ref, invdeg_ref, hlt_ref, hself_ref):
    hlt = lax.dot_general(w_ref[...].astype(jnp.bfloat16),
                          ht_ref[...].astype(jnp.bfloat16),
                          (((0,), (0,)), ((), ())),
                          preferred_element_type=_f32)
    hlt_ref[...] = hlt
    hself_ref[...] = hlt * invdeg_ref[...]


def _final_body(h3t_ref, batch_ref, wl_ref, bl_ref, wl2_ref, bl2_ref,
                out_ref, acc_ref, cnt_ref):
    i = pl.program_id(0)
    bt = batch_ref[...]
    oh = (lax.broadcasted_iota(_i32, (G, TB), 0) == bt).astype(_f32)
    ps = lax.dot_general(h3t_ref[...], oh, (((1,), (1,)), ((), ())),
                         precision=lax.Precision.HIGHEST,
                         preferred_element_type=_f32)
    cs = lax.dot_general(jnp.ones((1, TB), _f32), oh, (((1,), (1,)), ((), ())),
                         precision=lax.Precision.HIGHEST,
                         preferred_element_type=_f32)

    @pl.when(i == 0)
    def _():
        acc_ref[...] = jnp.zeros_like(acc_ref)
        cnt_ref[...] = jnp.zeros_like(cnt_ref)

    acc_ref[...] += ps
    cnt_ref[...] += cs

    @pl.when(i == NP // TB - 1)
    def _():
        pooled = acc_ref[...] / jnp.maximum(cnt_ref[...], 1.0)
        t = lax.dot_general(pooled.astype(jnp.bfloat16),
                            wl_ref[...].astype(jnp.bfloat16),
                            (((0,), (0,)), ((), ())),
                            preferred_element_type=_f32)
        t = jnp.maximum(t + bl_ref[...], 0.0)
        o = lax.dot_general(t.astype(jnp.bfloat16),
                            wl2_ref[...].astype(jnp.bfloat16),
                            (((1,), (0,)), ((), ())),
                            preferred_element_type=_f32)
        out_ref[...] = o + bl2_ref[...]


def kernel(x, edge_index, edge_weight, batch,
           W1, b1, W2, b2, W3, b3, Wl, bl, Wl2, bl2):
    row = edge_index[0]
    col = edge_index[1]
    xP = jnp.zeros((NP, D), _f32).at[:N].set(x)
    batchP = jnp.concatenate([batch.astype(_i32), jnp.full((NP - N,), G, _i32)])

    deg_parts = _deg_kernel(col, edge_weight)

    nb = NP // TB
    h1t, hself1, dinv2d, invdeg2d = pl.pallas_call(
        _prep_body,
        grid=(nb,),
        in_specs=[
            pl.BlockSpec((TB, D), lambda i: (i, 0)),
            pl.BlockSpec((D, F), lambda i: (0, 0)),
            pl.BlockSpec((NW, TB), lambda i: (0, i)),
        ],
        out_specs=[
            pl.BlockSpec((F, TB), lambda i: (0, i)),
            pl.BlockSpec((F, TB), lambda i: (0, i)),
            pl.BlockSpec((1, TB), lambda i: (0, i)),
            pl.BlockSpec((1, TB), lambda i: (0, i)),
        ],
        out_shape=[
            jax.ShapeDtypeStruct((F, NP), _f32),
            jax.ShapeDtypeStruct((F, NP), _f32),
            jax.ShapeDtypeStruct((1, NP), _f32),
            jax.ShapeDtypeStruct((1, NP), _f32),
        ],
    )(xP, W1, deg_parts)

    norm, packed = _norm_kernel(row, col, edge_weight, jnp.reshape(dinv2d, (NP,)))

    def mid_matmul(ht, W):
        return pl.pallas_call(
            _mid_body,
            grid=(nb,),
            in_specs=[
                pl.BlockSpec((F, TB), lambda i: (0, i)),
                pl.BlockSpec((F, F), lambda i: (0, 0)),
                pl.BlockSpec((1, TB), lambda i: (0, i)),
            ],
            out_specs=[
                pl.BlockSpec((F, TB), lambda i: (0, i)),
                pl.BlockSpec((F, TB), lambda i: (0, i)),
            ],
            out_shape=[
                jax.ShapeDtypeStruct((F, NP), _f32),
                jax.ShapeDtypeStruct((F, NP), _f32),
            ],
        )(ht, W, invdeg2d)

    h2t = _agg_kernel(h1t, hself1, packed, norm, b1)
    hlt2, hself2 = mid_matmul(h2t, W2)
    h3t = _agg_kernel(hlt2, hself2, packed, norm, b2)
    hlt3, hself3 = mid_matmul(h3t, W3)
    h4t = _agg_kernel(hlt3, hself3, packed, norm, b3)

    out = pl.pallas_call(
        _final_body,
        grid=(nb,),
        in_specs=[
            pl.BlockSpec((F, TB), lambda i: (0, i)),
            pl.BlockSpec((1, TB), lambda i: (0, i)),
            pl.BlockSpec((F, 32), lambda i: (0, 0)),
            pl.BlockSpec((1, 32), lambda i: (0, 0)),
            pl.BlockSpec((32, 1), lambda i: (0, 0)),
            pl.BlockSpec((1, 1), lambda i: (0, 0)),
        ],
        out_specs=pl.BlockSpec((G, 1), lambda i: (0, 0)),
        out_shape=jax.ShapeDtypeStruct((G, 1), _f32),
        scratch_shapes=[
            pltpu.VMEM((F, G), _f32),
            pltpu.VMEM((1, G), _f32),
        ],
    )(h4t, jnp.reshape(batchP, (1, NP)), Wl,
      jnp.reshape(bl, (1, 32)), Wl2, jnp.reshape(bl2, (1, 1)))
    return out
